# baseline (speedup 1.0000x reference)
import functools

import jax
import jax.numpy as jnp
from jax import lax
from jax.experimental import pallas as pl
from jax.experimental.pallas import tpu as pltpu

N = 10000
E = 320000
D = 128
DE = 16
NG = 64

_LRELU = 0.01


def _lrelu(x):
    return jnp.where(x >= 0, x, _LRELU * x)


def _elu(x):
    return jnp.where(x > 0, x, jnp.exp(jnp.minimum(x, 0.0)) - 1.0)


def _edge_ffn_body(x_ref, w1_ref, b1_ref, w2_ref, b2_ref, o_ref):
    h = _lrelu(jnp.dot(x_ref[...], w1_ref[...],
                       preferred_element_type=jnp.float32) + b1_ref[...])
    h = _lrelu(jnp.dot(h, w2_ref[...],
                       preferred_element_type=jnp.float32) + b2_ref[...])
    o_ref[...] = h


def _edge_ffn(edge_attr, E_W1, E_b1, E_W2, E_b2):
    eye8 = jnp.eye(8, dtype=jnp.float32)
    W1k = jnp.kron(eye8, E_W1)
    W2k = jnp.kron(eye8, E_W2)
    b1k = jnp.tile(E_b1, 8)[None, :]
    b2k = jnp.tile(E_b2, 8)[None, :]
    x2 = edge_attr.reshape(E // 8, 128)
    BLK = 2000
    grid = (E // 8) // BLK
    out = pl.pallas_call(
        _edge_ffn_body,
        grid=(grid,),
        in_specs=[
            pl.BlockSpec((BLK, 128), lambda i: (i, 0)),
            pl.BlockSpec((128, 128), lambda i: (0, 0)),
            pl.BlockSpec((1, 128), lambda i: (0, 0)),
            pl.BlockSpec((128, 128), lambda i: (0, 0)),
            pl.BlockSpec((1, 128), lambda i: (0, 0)),
        ],
        out_specs=pl.BlockSpec((BLK, 128), lambda i: (i, 0)),
        out_shape=jax.ShapeDtypeStruct((E // 8, 128), jnp.float32),
    )(x2, W1k, b1k, W2k, b2k)
    return out.reshape(E, DE)


def _node_ffn_body(x_ref, w1_ref, b1_ref, w2_ref, b2_ref, o_ref):
    h = _lrelu(jnp.dot(x_ref[...], w1_ref[...],
                       preferred_element_type=jnp.float32) + b1_ref[...])
    h = _lrelu(jnp.dot(h, w2_ref[...],
                       preferred_element_type=jnp.float32) + b2_ref[...])
    o_ref[...] = h


def _node_ffn(h, V_W1, V_b1, V_W2, V_b2):
    BLK = 2000
    grid = N // BLK
    return pl.pallas_call(
        _node_ffn_body,
        grid=(grid,),
        in_specs=[
            pl.BlockSpec((BLK, 128), lambda i: (i, 0)),
            pl.BlockSpec((128, 128), lambda i: (0, 0)),
            pl.BlockSpec((1, 128), lambda i: (0, 0)),
            pl.BlockSpec((128, 128), lambda i: (0, 0)),
            pl.BlockSpec((1, 128), lambda i: (0, 0)),
        ],
        out_specs=pl.BlockSpec((BLK, 128), lambda i: (i, 0)),
        out_shape=jax.ShapeDtypeStruct((N, 128), jnp.float32),
    )(h, V_W1, V_b1[None, :], V_W2, V_b2[None, :])


def _update_body_sum(a1a_ref, a1b_ref, a2a_ref, a2b_ref, hin_ref, hprev_ref, A1_ref,
                     A2_ref, A3_ref, b1_ref, W2_ref, b2_ref, wa_ref,
                     h_ref, hsum_ref):
    t = (jnp.dot(_elu(a1a_ref[...] + a1b_ref[...]), A1_ref[...],
                 preferred_element_type=jnp.float32)
         + jnp.dot(_elu(a2a_ref[...] + a2b_ref[...]), A2_ref[...],
                   preferred_element_type=jnp.float32)
         + jnp.dot(_elu(hin_ref[...]), A3_ref[...],
                   preferred_element_type=jnp.float32)
         + b1_ref[...])
    t = _lrelu(t)
    h = _lrelu(jnp.dot(t, W2_ref[...],
                       preferred_element_type=jnp.float32) + b2_ref[...])
    h = h * wa_ref[...]
    h_ref[...] = h
    hsum_ref[...] = h + hprev_ref[...]


def _update_body(a1a_ref, a1b_ref, a2a_ref, a2b_ref, hin_ref, A1_ref, A2_ref,
                 A3_ref, b1_ref, W2_ref, b2_ref, wa_ref, h_ref):
    t = (jnp.dot(_elu(a1a_ref[...] + a1b_ref[...]), A1_ref[...],
                 preferred_element_type=jnp.float32)
         + jnp.dot(_elu(a2a_ref[...] + a2b_ref[...]), A2_ref[...],
                   preferred_element_type=jnp.float32)
         + jnp.dot(_elu(hin_ref[...]), A3_ref[...],
                   preferred_element_type=jnp.float32)
         + b1_ref[...])
    t = _lrelu(t)
    h = _lrelu(jnp.dot(t, W2_ref[...],
                       preferred_element_type=jnp.float32) + b2_ref[...])
    h_ref[...] = h * wa_ref[...]


def _update(a1p, a2p, hin, hprev, A1, A2, A3, b1, W2, b2, wa2d):
    BLK = 2000
    grid = N // BLK
    row = lambda i: (i, 0)
    fix = lambda i: (0, 0)
    in_specs = [
        pl.BlockSpec((BLK, 128), row),
        pl.BlockSpec((BLK, 128), row),
        pl.BlockSpec((BLK, DE), row),
        pl.BlockSpec((BLK, DE), row),
        pl.BlockSpec((BLK, 128), row),
    ]
    args = [a1p[0], a1p[1], a2p[0], a2p[1], hin]
    if hprev is not None:
        in_specs.append(pl.BlockSpec((BLK, 128), row))
        args.append(hprev)
    in_specs += [
        pl.BlockSpec((128, 128), fix),
        pl.BlockSpec((DE, 128), fix),
        pl.BlockSpec((128, 128), fix),
        pl.BlockSpec((1, 128), fix),
        pl.BlockSpec((128, 128), fix),
        pl.BlockSpec((1, 128), fix),
        pl.BlockSpec((BLK, 128), row),
    ]
    args += [A1, A2, A3, b1[None, :], W2, b2[None, :], wa2d]
    if hprev is not None:
        return pl.pallas_call(
            _update_body_sum,
            grid=(grid,),
            in_specs=in_specs,
            out_specs=[pl.BlockSpec((BLK, 128), row)] * 2,
            out_shape=[jax.ShapeDtypeStruct((N, 128), jnp.float32)] * 2,
        )(*args)
    return pl.pallas_call(
        _update_body,
        grid=(grid,),
        in_specs=in_specs,
        out_specs=pl.BlockSpec((BLK, 128), row),
        out_shape=jax.ShapeDtypeStruct((N, 128), jnp.float32),
    )(*args)


def _readout_body(h3_ref, x_ref, RW_ref, rb_ref, scale_ref, beta_ref,
                  batch_ref, o_ref, sums_ref, counts_ref):
    i = pl.program_id(0)

    @pl.when(i == 0)
    def _init():
        sums_ref[...] = jnp.zeros_like(sums_ref)
        counts_ref[...] = jnp.zeros_like(counts_ref)

    r = h3_ref[...] + x_ref[...]
    r = jnp.dot(r, RW_ref[...], preferred_element_type=jnp.float32) + rb_ref[...]
    r = r * scale_ref[...] + beta_ref[...]
    r = _lrelu(r)
    bids = batch_ref[0, 0, :]
    blk = r.shape[0]
    mask = (bids[None, :] == lax.broadcasted_iota(jnp.int32, (NG, blk), 0))
    maskf = mask.astype(jnp.float32)
    sums_ref[...] += jnp.dot(maskf, r, preferred_element_type=jnp.float32)
    counts_ref[...] += jnp.sum(maskf, axis=1, keepdims=True)

    @pl.when(i == pl.num_programs(0) - 1)
    def _fin():
        o_ref[...] = sums_ref[...] / jnp.maximum(counts_ref[...], 1.0)


def _readout(h3, x, R_W, R_b, scale, beta, batch):
    BLK = 1000
    grid = N // BLK
    batch3 = batch.reshape(grid, 1, BLK).astype(jnp.int32)
    row = lambda i: (i, 0)
    fix = lambda i: (0, 0)
    return pl.pallas_call(
        _readout_body,
        grid=(grid,),
        in_specs=[
            pl.BlockSpec((BLK, 128), row),
            pl.BlockSpec((BLK, 128), row),
            pl.BlockSpec((128, 128), fix),
            pl.BlockSpec((1, 128), fix),
            pl.BlockSpec((1, 128), fix),
            pl.BlockSpec((1, 128), fix),
            pl.BlockSpec((1, 1, BLK), lambda i: (i, 0, 0)),
        ],
        out_specs=pl.BlockSpec((NG, 128), fix),
        out_shape=jax.ShapeDtypeStruct((NG, 128), jnp.float32),
        scratch_shapes=[
            pltpu.VMEM((NG, 128), jnp.float32),
            pltpu.VMEM((NG, 1), jnp.float32),
        ],
    )(h3, x, R_W, R_b[None, :], scale[None, :], beta[None, :], batch3)


NW = 32
EW = E // NW
CH = 80
NP = 5
PCH = 25
PE = PCH * CH
RPT = 624


def _sc_mesh():
    from jax.experimental.pallas import tpu_sc as plsc
    return plsc.VectorSubcoreMesh(core_axis_name="c", subcore_axis_name="s",
                                  num_cores=2, num_subcores=16)


def _make_gather_scatter(F, gather):
    from jax.experimental.pallas import tpu_sc as plsc
    KC = F // 16

    def body(tab, src_h, dst_h, wb_h, out, src_v, dst_v, wb_v, rows_a,
             rows_b, rows_c, acc, sga, sgb, sgc, ssa, ssb, ssc):
        cid = lax.axis_index("c")
        sid = lax.axis_index("s")
        wid = sid * 2 + cid

        zeros16 = jnp.zeros((16,), jnp.float32)

        def zrow(e, car):
            for k in range(KC):
                rows_a[e, pl.ds(k * 16, 16)] = zeros16
            return car

        lax.fori_loop(0, CH, zrow, 0)
        base = pl.multiple_of(sid * RPT, 8)
        for j in range(7):
            pltpu.sync_copy(rows_a, acc.at[pl.ds(base + j * CH, CH)])
        pltpu.sync_copy(rows_a.at[pl.ds(0, 64)],
                        acc.at[pl.ds(base + 7 * CH, 64)])

        @pl.when(sid == 15)
        def _ztail():
            pltpu.sync_copy(rows_a.at[pl.ds(0, N - 16 * RPT)],
                            acc.at[pl.ds(16 * RPT, N - 16 * RPT)])

        plsc.subcore_barrier()

        rows = [rows_a, rows_b, rows_c]
        gsems = [sga, sgb, sgc]
        ssems = [ssa, ssb, ssc]

        def ppass(p, carp):
            if gather:
                pltpu.sync_copy(src_h.at[wid, p], src_v)
            pltpu.sync_copy(dst_h.at[wid, p], dst_v)
            pltpu.sync_copy(wb_h.at[wid, p], wb_v)

            def gsrc(c):
                if gather:
                    return tab.at[src_v.at[c]]
                return tab.at[pl.ds(wid * EW + p * PE + c * CH, CH)]

            def start_g(c, b):
                pltpu.async_copy(gsrc(c), rows[b], gsems[b])

            def wait_g(c, b):
                pltpu.make_async_copy(gsrc(c), rows[b], gsems[b]).wait()

            def start_s(c, b):
                pltpu.async_copy(rows[b], acc.at[dst_v.at[c]], ssems[b],
                                 add=True)

            def wait_s(b):
                pltpu.make_async_copy(rows[b], acc.at[dst_v.at[0]],
                                      ssems[b]).wait()

            def scale(c, b):
                buf = rows[b]

                def grp(g, car2):
                    wbv = wb_v[c, pl.ds(g * 16, 16)]
                    for j in range(16):
                        w = wbv[j]
                        e = g * 16 + j
                        for k in range(KC):
                            sl = pl.ds(k * 16, 16)
                            buf[e, sl] = buf[e, sl] * w
                    return car2

                lax.fori_loop(0, CH // 16, grp, 0)

            start_g(0, 0)
            start_g(1, 1)
            wait_g(0, 0)
            scale(0, 0)
            start_s(0, 0)
            start_g(2, 2)
            wait_g(1, 1)
            scale(1, 1)
            start_s(1, 1)
            wait_s(0)
            start_g(3, 0)

            def triple(t, car):
                c0 = 3 * t + 2
                for i, b in enumerate((2, 0, 1)):
                    c = c0 + i
                    wait_g(c, b)
                    scale(c, b)
                    start_s(c, b)
                    wait_s((b + 2) % 3)
                    start_g(c + 2, (b + 2) % 3)
                return car

            lax.fori_loop(0, (PCH - 4) // 3, triple, 0)
            wait_g(PCH - 2, 2)
            scale(PCH - 2, 2)
            start_s(PCH - 2, 2)
            wait_g(PCH - 1, 0)
            scale(PCH - 1, 0)
            start_s(PCH - 1, 0)
            wait_s(1)
            wait_s(2)
            wait_s(0)
            return carp

        lax.fori_loop(0, NP, ppass, 0)
        plsc.subcore_barrier()
        wbase = pl.multiple_of(sid * RPT, 8)
        pltpu.sync_copy(acc.at[pl.ds(wbase, RPT)],
                        out.at[cid, pl.ds(wbase, RPT)])

        @pl.when(sid == 15)
        def _wtail():
            pltpu.sync_copy(acc.at[pl.ds(16 * RPT, N - 16 * RPT)],
                            out.at[cid, pl.ds(16 * RPT, N - 16 * RPT)])

    return pl.kernel(
        body,
        out_type=jax.ShapeDtypeStruct((2, N, F), jnp.float32),
        mesh=_sc_mesh(),
        compiler_params=pltpu.CompilerParams(use_tc_tiling_on_sc=(F == D)),
        scratch_types=[
            pltpu.VMEM((PCH, CH), jnp.int32) if gather else
            pltpu.VMEM((1, 1), jnp.int32),
            pltpu.VMEM((PCH, CH), jnp.int32),
            pltpu.VMEM((PCH, CH), jnp.float32),
            pltpu.VMEM((CH, F), jnp.float32),
            pltpu.VMEM((CH, F), jnp.float32),
            pltpu.VMEM((CH, F), jnp.float32),
            pltpu.VMEM_SHARED((N, F), jnp.float32),
            pltpu.SemaphoreType.DMA,
            pltpu.SemaphoreType.DMA,
            pltpu.SemaphoreType.DMA,
            pltpu.SemaphoreType.DMA,
            pltpu.SemaphoreType.DMA,
            pltpu.SemaphoreType.DMA,
        ],
    )


def _aggr_edges(Eh, wb, dst):
    f = _make_gather_scatter(DE, gather=False)
    return f(Eh,
             jnp.zeros((NW, NP, 1, 1), jnp.int32),
             dst.reshape(NW, NP, PCH, CH),
             wb.reshape(NW, NP, PCH, CH))


def _aggr_nodes(Vh, src, dst, wb):
    f = _make_gather_scatter(D, gather=True)
    return f(Vh, src.reshape(NW, NP, PCH, CH), dst.reshape(NW, NP, PCH, CH),
             wb.reshape(NW, NP, PCH, CH))


def kernel(x, edge_index, edge_attr, w_atoms, w_bonds, batch,
           V_W1, V_b1, V_W2, V_b2,
           E_W1, E_b1, E_W2, E_b2,
           U_W1, U_b1, U_W2, U_b2,
           R_W, R_b, R_gamma, R_beta):
    src = edge_index[0].astype(jnp.int32)
    dst = edge_index[1].astype(jnp.int32)

    A1 = U_W1[:128]
    A2 = U_W1[128:144]
    A3 = U_W1[144:]
    wa2d = jnp.broadcast_to(w_atoms[:, None], (N, 128))
    bn_scale = R_gamma / jnp.sqrt(1.0 + 1e-5)

    Eh = _edge_ffn(edge_attr, E_W1, E_b1, E_W2, E_b2)
    aggr2 = _aggr_edges(Eh, w_bonds, dst)

    def layer(hin, hprev):
        Vh = _node_ffn(hin, V_W1, V_b1, V_W2, V_b2)
        a1p = _aggr_nodes(Vh, src, dst, w_bonds)
        return _update(a1p, aggr2, hin, hprev, A1, A2, A3, U_b1, U_W2, U_b2,
                       wa2d)

    h1 = layer(x, None)
    h2, hin2 = layer(h1, h1)
    h3 = layer(hin2, None)
    return _readout(h3, x, R_W, R_b, bn_scale, R_beta, batch)


# speedup vs baseline: 1.0510x; 1.0510x over previous
import functools

import jax
import jax.numpy as jnp
from jax import lax
from jax.experimental import pallas as pl
from jax.experimental.pallas import tpu as pltpu

N = 10000
E = 320000
D = 128
DE = 16
NG = 64

_LRELU = 0.01


def _lrelu(x):
    return jnp.where(x >= 0, x, _LRELU * x)


def _elu(x):
    return jnp.where(x > 0, x, jnp.exp(jnp.minimum(x, 0.0)) - 1.0)


def _edge_ffn_body(x_ref, w1_ref, b1_ref, w2_ref, b2_ref, o_ref):
    h = _lrelu(jnp.dot(x_ref[...], w1_ref[...],
                       preferred_element_type=jnp.float32) + b1_ref[...])
    h = _lrelu(jnp.dot(h, w2_ref[...],
                       preferred_element_type=jnp.float32) + b2_ref[...])
    o_ref[...] = h


def _edge_ffn(edge_attr, E_W1, E_b1, E_W2, E_b2):
    eye8 = jnp.eye(8, dtype=jnp.float32)
    W1k = jnp.kron(eye8, E_W1)
    W2k = jnp.kron(eye8, E_W2)
    b1k = jnp.tile(E_b1, 8)[None, :]
    b2k = jnp.tile(E_b2, 8)[None, :]
    x2 = edge_attr.reshape(E // 8, 128)
    BLK = 2000
    grid = (E // 8) // BLK
    out = pl.pallas_call(
        _edge_ffn_body,
        grid=(grid,),
        in_specs=[
            pl.BlockSpec((BLK, 128), lambda i: (i, 0)),
            pl.BlockSpec((128, 128), lambda i: (0, 0)),
            pl.BlockSpec((1, 128), lambda i: (0, 0)),
            pl.BlockSpec((128, 128), lambda i: (0, 0)),
            pl.BlockSpec((1, 128), lambda i: (0, 0)),
        ],
        out_specs=pl.BlockSpec((BLK, 128), lambda i: (i, 0)),
        out_shape=jax.ShapeDtypeStruct((E // 8, 128), jnp.float32),
    )(x2, W1k, b1k, W2k, b2k)
    return out.reshape(E, DE)


def _node_ffn_body(x_ref, w1_ref, b1_ref, w2_ref, b2_ref, o_ref):
    h = _lrelu(jnp.dot(x_ref[...], w1_ref[...],
                       preferred_element_type=jnp.float32) + b1_ref[...])
    h = _lrelu(jnp.dot(h, w2_ref[...],
                       preferred_element_type=jnp.float32) + b2_ref[...])
    o_ref[...] = h


def _node_ffn(h, V_W1, V_b1, V_W2, V_b2):
    BLK = 2000
    grid = N // BLK
    return pl.pallas_call(
        _node_ffn_body,
        grid=(grid,),
        in_specs=[
            pl.BlockSpec((BLK, 128), lambda i: (i, 0)),
            pl.BlockSpec((128, 128), lambda i: (0, 0)),
            pl.BlockSpec((1, 128), lambda i: (0, 0)),
            pl.BlockSpec((128, 128), lambda i: (0, 0)),
            pl.BlockSpec((1, 128), lambda i: (0, 0)),
        ],
        out_specs=pl.BlockSpec((BLK, 128), lambda i: (i, 0)),
        out_shape=jax.ShapeDtypeStruct((N, 128), jnp.float32),
    )(h, V_W1, V_b1[None, :], V_W2, V_b2[None, :])


def _ffn_in_body(h, w1_ref, b1_ref, w2_ref, b2_ref):
    t = _lrelu(jnp.dot(h, w1_ref[...],
                       preferred_element_type=jnp.float32) + b1_ref[...])
    return _lrelu(jnp.dot(t, w2_ref[...],
                          preferred_element_type=jnp.float32) + b2_ref[...])


def _update_core(a1a_ref, a1b_ref, a2a_ref, a2b_ref, hin_ref, A1_ref, A2_ref,
                 A3_ref, b1_ref, W2_ref, b2_ref, wa_ref):
    t = (jnp.dot(_elu(a1a_ref[...] + a1b_ref[...]), A1_ref[...],
                 preferred_element_type=jnp.float32)
         + jnp.dot(_elu(a2a_ref[...] + a2b_ref[...]), A2_ref[...],
                   preferred_element_type=jnp.float32)
         + jnp.dot(_elu(hin_ref[...]), A3_ref[...],
                   preferred_element_type=jnp.float32)
         + b1_ref[...])
    t = _lrelu(t)
    h = _lrelu(jnp.dot(t, W2_ref[...],
                       preferred_element_type=jnp.float32) + b2_ref[...])
    return h * wa_ref[...]


def _ufirst_body(a1a, a1b, a2a, a2b, hin, A1, A2, A3, b1, W2, b2, wa,
                 vw1, vb1, vw2, vb2, h_ref, vh_ref):
    h = _update_core(a1a, a1b, a2a, a2b, hin, A1, A2, A3, b1, W2, b2, wa)
    h_ref[...] = h
    vh_ref[...] = _ffn_in_body(h, vw1, vb1, vw2, vb2)


def _umid_body(a1a, a1b, a2a, a2b, hin, A1, A2, A3, b1, W2, b2, wa,
               vw1, vb1, vw2, vb2, hs_ref, vh_ref):
    h = _update_core(a1a, a1b, a2a, a2b, hin, A1, A2, A3, b1, W2, b2, wa)
    s = h + hin[...]
    hs_ref[...] = s
    vh_ref[...] = _ffn_in_body(s, vw1, vb1, vw2, vb2)


def _ulast_body(a1a, a1b, a2a, a2b, hin, A1, A2, A3, b1, W2, b2, wa,
                x_ref, RW_ref, rb_ref, scale_ref, beta_ref, batch_ref,
                o_ref, sums_ref, counts_ref):
    i = pl.program_id(0)

    @pl.when(i == 0)
    def _init():
        sums_ref[...] = jnp.zeros_like(sums_ref)
        counts_ref[...] = jnp.zeros_like(counts_ref)

    h = _update_core(a1a, a1b, a2a, a2b, hin, A1, A2, A3, b1, W2, b2, wa)
    r = h + x_ref[...]
    r = jnp.dot(r, RW_ref[...], preferred_element_type=jnp.float32) + rb_ref[...]
    r = r * scale_ref[...] + beta_ref[...]
    r = _lrelu(r)
    bids = batch_ref[0, 0, :]
    blk = r.shape[0]
    mask = (bids[None, :] == lax.broadcasted_iota(jnp.int32, (NG, blk), 0))
    maskf = mask.astype(jnp.float32)
    sums_ref[...] += jnp.dot(maskf, r, preferred_element_type=jnp.float32)
    counts_ref[...] += jnp.sum(maskf, axis=1, keepdims=True)

    @pl.when(i == pl.num_programs(0) - 1)
    def _fin():
        o_ref[...] = sums_ref[...] / jnp.maximum(counts_ref[...], 1.0)


_UBLK = 2000
_row = lambda i: (i, 0)
_fix = lambda i: (0, 0)

_USPECS = [
    pl.BlockSpec((_UBLK, 128), _row),
    pl.BlockSpec((_UBLK, 128), _row),
    pl.BlockSpec((_UBLK, DE), _row),
    pl.BlockSpec((_UBLK, DE), _row),
    pl.BlockSpec((_UBLK, 128), _row),
    pl.BlockSpec((128, 128), _fix),
    pl.BlockSpec((DE, 128), _fix),
    pl.BlockSpec((128, 128), _fix),
    pl.BlockSpec((1, 128), _fix),
    pl.BlockSpec((128, 128), _fix),
    pl.BlockSpec((1, 128), _fix),
    pl.BlockSpec((_UBLK, 128), _row),
]

_VSPECS = [
    pl.BlockSpec((128, 128), _fix),
    pl.BlockSpec((1, 128), _fix),
    pl.BlockSpec((128, 128), _fix),
    pl.BlockSpec((1, 128), _fix),
]


def _update_layer(body, a1p, a2p, hin, uargs, vargs):
    return pl.pallas_call(
        body,
        grid=(N // _UBLK,),
        in_specs=_USPECS + _VSPECS,
        out_specs=[pl.BlockSpec((_UBLK, 128), _row)] * 2,
        out_shape=[jax.ShapeDtypeStruct((N, 128), jnp.float32)] * 2,
    )(a1p[0], a1p[1], a2p[0], a2p[1], hin, *uargs, *vargs)


def _update_last(a1p, a2p, hin, uargs, x, R_W, R_b, scale, beta, batch):
    batch3 = batch.reshape(N // _UBLK, 1, _UBLK).astype(jnp.int32)
    return pl.pallas_call(
        _ulast_body,
        grid=(N // _UBLK,),
        in_specs=_USPECS + [
            pl.BlockSpec((_UBLK, 128), _row),
            pl.BlockSpec((128, 128), _fix),
            pl.BlockSpec((1, 128), _fix),
            pl.BlockSpec((1, 128), _fix),
            pl.BlockSpec((1, 128), _fix),
            pl.BlockSpec((1, 1, _UBLK), lambda i: (i, 0, 0)),
        ],
        out_specs=pl.BlockSpec((NG, 128), _fix),
        out_shape=jax.ShapeDtypeStruct((NG, 128), jnp.float32),
        scratch_shapes=[
            pltpu.VMEM((NG, 128), jnp.float32),
            pltpu.VMEM((NG, 1), jnp.float32),
        ],
    )(a1p[0], a1p[1], a2p[0], a2p[1], hin, *uargs,
      x, R_W, R_b[None, :], scale[None, :], beta[None, :], batch3)


NW = 32
EW = E // NW
CH = 80
NP = 5
PCH = 25
PE = PCH * CH
RPT = 624


def _sc_mesh():
    from jax.experimental.pallas import tpu_sc as plsc
    return plsc.VectorSubcoreMesh(core_axis_name="c", subcore_axis_name="s",
                                  num_cores=2, num_subcores=16)


def _make_gather_scatter(F, gather):
    from jax.experimental.pallas import tpu_sc as plsc
    KC = F // 16

    def body(tab, src_h, dst_h, wb_h, out, src_v, dst_v, wb_v, rows_a,
             rows_b, rows_c, acc, sga, sgb, sgc, ssa, ssb, ssc):
        cid = lax.axis_index("c")
        sid = lax.axis_index("s")
        wid = sid * 2 + cid

        zeros16 = jnp.zeros((16,), jnp.float32)

        def zrow(e, car):
            for k in range(KC):
                rows_a[e, pl.ds(k * 16, 16)] = zeros16
            return car

        lax.fori_loop(0, CH, zrow, 0)
        base = pl.multiple_of(sid * RPT, 8)
        for j in range(7):
            pltpu.sync_copy(rows_a, acc.at[pl.ds(base + j * CH, CH)])
        pltpu.sync_copy(rows_a.at[pl.ds(0, 64)],
                        acc.at[pl.ds(base + 7 * CH, 64)])

        @pl.when(sid == 15)
        def _ztail():
            pltpu.sync_copy(rows_a.at[pl.ds(0, N - 16 * RPT)],
                            acc.at[pl.ds(16 * RPT, N - 16 * RPT)])

        plsc.subcore_barrier()

        rows = [rows_a, rows_b, rows_c]
        gsems = [sga, sgb, sgc]
        ssems = [ssa, ssb, ssc]

        def ppass(p, carp):
            if gather:
                pltpu.sync_copy(src_h.at[wid, p], src_v)
            pltpu.sync_copy(dst_h.at[wid, p], dst_v)
            pltpu.sync_copy(wb_h.at[wid, p], wb_v)

            def gsrc(c):
                if gather:
                    return tab.at[src_v.at[c]]
                return tab.at[pl.ds(wid * EW + p * PE + c * CH, CH)]

            def start_g(c, b):
                pltpu.async_copy(gsrc(c), rows[b], gsems[b])

            def wait_g(c, b):
                pltpu.make_async_copy(gsrc(c), rows[b], gsems[b]).wait()

            def start_s(c, b):
                pltpu.async_copy(rows[b], acc.at[dst_v.at[c]], ssems[b],
                                 add=True)

            def wait_s(b):
                pltpu.make_async_copy(rows[b], acc.at[dst_v.at[0]],
                                      ssems[b]).wait()

            def scale(c, b):
                buf = rows[b]

                def grp(g, car2):
                    wbv = wb_v[c, pl.ds(g * 16, 16)]
                    for j in range(16):
                        w = wbv[j]
                        e = g * 16 + j
                        for k in range(KC):
                            sl = pl.ds(k * 16, 16)
                            buf[e, sl] = buf[e, sl] * w
                    return car2

                lax.fori_loop(0, CH // 16, grp, 0)

            start_g(0, 0)
            start_g(1, 1)
            wait_g(0, 0)
            scale(0, 0)
            start_s(0, 0)
            start_g(2, 2)
            wait_g(1, 1)
            scale(1, 1)
            start_s(1, 1)
            wait_s(0)
            start_g(3, 0)

            def triple(t, car):
                c0 = 3 * t + 2
                for i, b in enumerate((2, 0, 1)):
                    c = c0 + i
                    wait_g(c, b)
                    scale(c, b)
                    start_s(c, b)
                    wait_s((b + 2) % 3)
                    start_g(c + 2, (b + 2) % 3)
                return car

            lax.fori_loop(0, (PCH - 4) // 3, triple, 0)
            wait_g(PCH - 2, 2)
            scale(PCH - 2, 2)
            start_s(PCH - 2, 2)
            wait_g(PCH - 1, 0)
            scale(PCH - 1, 0)
            start_s(PCH - 1, 0)
            wait_s(1)
            wait_s(2)
            wait_s(0)
            return carp

        lax.fori_loop(0, NP, ppass, 0)
        plsc.subcore_barrier()
        wbase = pl.multiple_of(sid * RPT, 8)
        pltpu.sync_copy(acc.at[pl.ds(wbase, RPT)],
                        out.at[cid, pl.ds(wbase, RPT)])

        @pl.when(sid == 15)
        def _wtail():
            pltpu.sync_copy(acc.at[pl.ds(16 * RPT, N - 16 * RPT)],
                            out.at[cid, pl.ds(16 * RPT, N - 16 * RPT)])

    return pl.kernel(
        body,
        out_type=jax.ShapeDtypeStruct((2, N, F), jnp.float32),
        mesh=_sc_mesh(),
        compiler_params=pltpu.CompilerParams(use_tc_tiling_on_sc=False),
        scratch_types=[
            pltpu.VMEM((PCH, CH), jnp.int32) if gather else
            pltpu.VMEM((1, 1), jnp.int32),
            pltpu.VMEM((PCH, CH), jnp.int32),
            pltpu.VMEM((PCH, CH), jnp.float32),
            pltpu.VMEM((CH, F), jnp.float32),
            pltpu.VMEM((CH, F), jnp.float32),
            pltpu.VMEM((CH, F), jnp.float32),
            pltpu.VMEM_SHARED((N, F), jnp.float32),
            pltpu.SemaphoreType.DMA,
            pltpu.SemaphoreType.DMA,
            pltpu.SemaphoreType.DMA,
            pltpu.SemaphoreType.DMA,
            pltpu.SemaphoreType.DMA,
            pltpu.SemaphoreType.DMA,
        ],
    )


def _aggr_edges(Eh, wb, dst):
    f = _make_gather_scatter(DE, gather=False)
    return f(Eh,
             jnp.zeros((NW, NP, 1, 1), jnp.int32),
             dst.reshape(NW, NP, PCH, CH),
             wb.reshape(NW, NP, PCH, CH))


def _aggr_nodes(Vh, src, dst, wb):
    f = _make_gather_scatter(D, gather=True)
    return f(Vh, src.reshape(NW, NP, PCH, CH), dst.reshape(NW, NP, PCH, CH),
             wb.reshape(NW, NP, PCH, CH))


def kernel(x, edge_index, edge_attr, w_atoms, w_bonds, batch,
           V_W1, V_b1, V_W2, V_b2,
           E_W1, E_b1, E_W2, E_b2,
           U_W1, U_b1, U_W2, U_b2,
           R_W, R_b, R_gamma, R_beta):
    src = edge_index[0].astype(jnp.int32)
    dst = edge_index[1].astype(jnp.int32)

    wa2d = jnp.broadcast_to(w_atoms[:, None], (N, 128))
    uargs = (U_W1[:128], U_W1[128:144], U_W1[144:], U_b1[None, :],
             U_W2, U_b2[None, :], wa2d)
    vargs = (V_W1, V_b1[None, :], V_W2, V_b2[None, :])
    bn_scale = R_gamma / jnp.sqrt(1.0 + 1e-5)

    Eh = _edge_ffn(edge_attr, E_W1, E_b1, E_W2, E_b2)
    a2p = _aggr_edges(Eh, w_bonds, dst)

    Vh = _node_ffn(x, V_W1, V_b1, V_W2, V_b2)
    a1p = _aggr_nodes(Vh, src, dst, w_bonds)
    h1, Vh = _update_layer(_ufirst_body, a1p, a2p, x, uargs, vargs)

    a1p = _aggr_nodes(Vh, src, dst, w_bonds)
    hin2, Vh = _update_layer(_umid_body, a1p, a2p, h1, uargs, vargs)

    a1p = _aggr_nodes(Vh, src, dst, w_bonds)
    return _update_last(a1p, a2p, hin2, uargs, x, R_W, R_b, bn_scale,
                        R_beta, batch)
